# baseline (speedup 1.0000x reference)
import functools

import jax
import jax.numpy as jnp
from jax import lax
from jax.experimental import pallas as pl
from jax.experimental.pallas import tpu as pltpu
from jax.experimental.pallas import tpu_sc as plsc

_BR = 512


def _vq_body(zz_ref, ee_ref, z_ref, e_ref,
             idx_ref, stats_ref,
             acc_ref, sq_ref):
    i = pl.program_id(0)
    nsteps = pl.num_programs(0)
    K = e_ref.shape[0]
    btot = nsteps * z_ref.shape[0]

    zb = z_ref[...]
    e = e_ref[...]
    zzb = zz_ref[...]
    ee = ee_ref[...]

    dots = lax.dot_general(zb, e, (((1,), (1,)), ((), ())),
                           preferred_element_type=jnp.float32)
    d = zzb + ee[None, :] - 2.0 * dots

    colf = lax.broadcasted_iota(jnp.int32, (1, K), 1).astype(jnp.float32)
    dmin = jnp.min(d, axis=1, keepdims=True)
    idxf = jnp.min(jnp.where(d == dmin, colf, jnp.float32(K)),
                   axis=1, keepdims=True)
    idx2 = idxf.astype(jnp.int32)

    p = jnp.exp(dmin - d)
    s1 = jnp.sum(p, axis=1)
    srow = (1.0 / s1)[None, :]
    soft_sum = lax.dot_general(srow, p, (((1,), (0,)), ((), ())),
                               preferred_element_type=jnp.float32)
    sq_c = jnp.sum(dmin)

    @pl.when(i == 0)
    def _init():
        acc_ref[...] = soft_sum
        sq_ref[0] = sq_c

    @pl.when(i > 0)
    def _accum():
        acc_ref[...] += soft_sum
        sq_ref[0] += sq_c

    idx_ref[...] = idx2[:, 0]

    @pl.when(i == nsteps - 1)
    def _finalize():
        avg = acc_ref[...] / btot
        ent = -jnp.sum(avg * jnp.log(avg + 1e-10))
        div = jnp.log(jnp.float32(K)) - ent
        sqm = sq_ref[0] / (btot * z_ref.shape[1])
        stats_ref[0] = sqm * 0.25 + sqm + 0.1 * div
        stats_ref[1] = div


def _vq_main(z, embedding, zz, ee, interpret=False):
    B, D = z.shape
    K = embedding.shape[0]
    grid = (B // _BR,)
    return pl.pallas_call(
        _vq_body,
        grid=grid,
        in_specs=[
            pl.BlockSpec((_BR, 1), lambda i: (i, 0)),
            pl.BlockSpec((K,), lambda i: (0,)),
            pl.BlockSpec((_BR, D), lambda i: (i, 0)),
            pl.BlockSpec((K, D), lambda i: (0, 0)),
        ],
        out_specs=[
            pl.BlockSpec((_BR,), lambda i: (i,)),
            pl.BlockSpec(memory_space=pltpu.SMEM),
        ],
        out_shape=[
            jax.ShapeDtypeStruct((B,), jnp.int32),
            jax.ShapeDtypeStruct((8,), jnp.float32),
        ],
        scratch_shapes=[
            pltpu.VMEM((1, K), jnp.float32),
            pltpu.SMEM((1,), jnp.float32),
        ],
        compiler_params=pltpu.CompilerParams(
            vmem_limit_bytes=110 * 1024 * 1024),
        interpret=interpret,
    )(zz, ee, z, embedding)


_NBIN_ROWS = 8


def _sc_gather_hist(emb_pad, idx2d, B, K):
    info = plsc.get_sparse_core_info()
    NC, NS = info.num_cores, info.num_subcores
    NW = NC * NS
    nrows = idx2d.shape[0]
    rows_per_w = nrows // NW
    CHUNK = idx2d.shape[1]
    L = info.num_lanes

    @functools.partial(
        pl.kernel,
        mesh=plsc.VectorSubcoreMesh(core_axis_name="c", subcore_axis_name="s"),
        out_type=[
            jax.ShapeDtypeStruct((B, 128), jnp.float32),
            jax.ShapeDtypeStruct((NW * _NBIN_ROWS * K,), jnp.float32),
        ],
        scratch_types=[
            pltpu.VMEM((rows_per_w, CHUNK), jnp.int32),
            pltpu.VMEM((CHUNK, 128), jnp.float32),
            pltpu.VMEM((_NBIN_ROWS * K,), jnp.float32),
            pltpu.SemaphoreType.DMA,
        ],
        compiler_params=pltpu.CompilerParams(needs_layout_passes=False),
    )
    def gather(table_hbm, idx_hbm, out_hbm, hist_hbm, idx_v, rows_v, bins_v,
               sem):
        wid = lax.axis_index("s") * NC + lax.axis_index("c")
        pltpu.sync_copy(idx_hbm.at[pl.ds(wid * rows_per_w, rows_per_w)], idx_v)

        zeros16 = jnp.zeros((L,), jnp.float32)

        def _zero(c, carry):
            bins_v[pl.ds(c * L, L)] = zeros16
            return carry

        lax.fori_loop(0, _NBIN_ROWS * K // L, _zero, 0)

        for j in range(rows_per_w):
            pltpu.async_copy(table_hbm.at[idx_v.at[j]], rows_v, sem).wait()
            pltpu.sync_copy(
                rows_v,
                out_hbm.at[pl.ds((wid * rows_per_w + j) * CHUNK, CHUNK)])

        lane = lax.iota(jnp.int32, L)
        base16 = jnp.bitwise_and(lane, _NBIN_ROWS - 1) * K
        ones16 = jnp.ones((L,), jnp.float32)
        lo = lane < _NBIN_ROWS
        hi = lane >= _NBIN_ROWS
        for j in range(rows_per_w):
            for g in range(CHUNK // L):
                idx16 = idx_v[j, pl.ds(g * L, L)] + base16
                plsc.addupdate_scatter(bins_v, [idx16], ones16, mask=lo)
                plsc.addupdate_scatter(bins_v, [idx16], ones16, mask=hi)

        pltpu.sync_copy(
            bins_v,
            hist_hbm.at[pl.ds(wid * _NBIN_ROWS * K, _NBIN_ROWS * K)])

    return gather(emb_pad, idx2d)


def _perp_body(parts_ref, out_ref):
    btot = 8192.0
    hist = jnp.sum(parts_ref[...], axis=0, keepdims=True)
    probs = hist / btot
    out_ref[0] = jnp.exp(-jnp.sum(probs * jnp.log(probs + 1e-10)))


def _perplexity(hist_parts):
    return pl.pallas_call(
        _perp_body,
        out_specs=pl.BlockSpec(memory_space=pltpu.SMEM),
        out_shape=jax.ShapeDtypeStruct((1,), jnp.float32),
    )(hist_parts)


def kernel(z, embedding):
    B, D = z.shape
    K = embedding.shape[0]
    zz = jnp.sum(z * z, axis=1, keepdims=True)
    ee = jnp.sum(embedding * embedding, axis=1)
    indices, stats = _vq_main(z, embedding, zz, ee)
    emb_pad = jnp.pad(embedding, ((0, 0), (0, 128 - D)))
    z_q_pad, hist_parts = _sc_gather_hist(
        emb_pad, indices.reshape(B // 128, 128), B, K)
    perp = _perplexity(hist_parts.reshape(-1, K))
    return (z_q_pad[:, :D], stats[0], perp[0], indices, stats[1])


# speedup vs baseline: 1.1704x; 1.1704x over previous
import functools

import jax
import jax.numpy as jnp
from jax import lax
from jax.experimental import pallas as pl
from jax.experimental.pallas import tpu as pltpu
from jax.experimental.pallas import tpu_sc as plsc

_BR = 512


def _vq_body(zz_ref, ee_ref, z_ref, e_ref,
             idx_ref, stats_ref,
             acc_ref, sq_ref):
    i = pl.program_id(0)
    nsteps = pl.num_programs(0)
    K = e_ref.shape[0]
    btot = nsteps * z_ref.shape[0]

    zb = z_ref[...]
    e = e_ref[...]
    zzb = zz_ref[...]
    ee = ee_ref[...]

    dots = lax.dot_general(zb, e, (((1,), (1,)), ((), ())),
                           preferred_element_type=jnp.float32)
    d = zzb + ee[None, :] - 2.0 * dots

    colf = lax.broadcasted_iota(jnp.int32, (1, K), 1).astype(jnp.float32)
    dmin = jnp.min(d, axis=1, keepdims=True)
    idxf = jnp.min(jnp.where(d == dmin, colf, jnp.float32(K)),
                   axis=1, keepdims=True)
    idx2 = idxf.astype(jnp.int32)

    p = jnp.exp(dmin - d)
    s1 = jnp.sum(p, axis=1)
    srow = (1.0 / s1)[None, :]
    soft_sum = lax.dot_general(srow, p, (((1,), (0,)), ((), ())),
                               preferred_element_type=jnp.float32)
    sq_c = jnp.sum(dmin)

    @pl.when(i == 0)
    def _init():
        acc_ref[...] = soft_sum
        sq_ref[0] = sq_c

    @pl.when(i > 0)
    def _accum():
        acc_ref[...] += soft_sum
        sq_ref[0] += sq_c

    idx_ref[...] = idx2[:, 0]

    @pl.when(i == nsteps - 1)
    def _finalize():
        avg = acc_ref[...] / btot
        ent = -jnp.sum(avg * jnp.log(avg + 1e-10))
        div = jnp.log(jnp.float32(K)) - ent
        sqm = sq_ref[0] / (btot * z_ref.shape[1])
        stats_ref[0] = sqm * 0.25 + sqm + 0.1 * div
        stats_ref[1] = div


def _vq_main(z, embedding, zz, ee, interpret=False):
    B, D = z.shape
    K = embedding.shape[0]
    grid = (B // _BR,)
    return pl.pallas_call(
        _vq_body,
        grid=grid,
        in_specs=[
            pl.BlockSpec((_BR, 1), lambda i: (i, 0)),
            pl.BlockSpec((K,), lambda i: (0,)),
            pl.BlockSpec((_BR, D), lambda i: (i, 0)),
            pl.BlockSpec((K, D), lambda i: (0, 0)),
        ],
        out_specs=[
            pl.BlockSpec((_BR,), lambda i: (i,)),
            pl.BlockSpec(memory_space=pltpu.SMEM),
        ],
        out_shape=[
            jax.ShapeDtypeStruct((B,), jnp.int32),
            jax.ShapeDtypeStruct((8,), jnp.float32),
        ],
        scratch_shapes=[
            pltpu.VMEM((1, K), jnp.float32),
            pltpu.SMEM((1,), jnp.float32),
        ],
        compiler_params=pltpu.CompilerParams(
            vmem_limit_bytes=110 * 1024 * 1024),
        interpret=interpret,
    )(zz, ee, z, embedding)


_NBIN_ROWS = 2


def _sc_gather_hist(emb_pad, idx2d, B, K):
    info = plsc.get_sparse_core_info()
    NC, NS = info.num_cores, info.num_subcores
    NW = NC * NS
    nrows = idx2d.shape[0]
    rows_per_w = nrows // NW
    CHUNK = idx2d.shape[1]
    L = info.num_lanes

    @functools.partial(
        pl.kernel,
        mesh=plsc.VectorSubcoreMesh(core_axis_name="c", subcore_axis_name="s"),
        out_type=[
            jax.ShapeDtypeStruct((B, 128), jnp.float32),
            jax.ShapeDtypeStruct((NW * K,), jnp.float32),
        ],
        scratch_types=[
            pltpu.VMEM((rows_per_w, CHUNK), jnp.int32),
            pltpu.VMEM((CHUNK, 128), jnp.float32),
            pltpu.VMEM((_NBIN_ROWS * K,), jnp.float32),
            pltpu.SemaphoreType.DMA,
        ],
        compiler_params=pltpu.CompilerParams(needs_layout_passes=False),
    )
    def gather(table_hbm, idx_hbm, out_hbm, hist_hbm, idx_v, rows_v, bins_v,
               sem):
        wid = lax.axis_index("s") * NC + lax.axis_index("c")
        pltpu.sync_copy(idx_hbm.at[pl.ds(wid * rows_per_w, rows_per_w)], idx_v)

        cp = pltpu.async_copy(table_hbm.at[idx_v.at[0]], rows_v, sem)

        zeros16 = jnp.zeros((L,), jnp.float32)
        UNROLL = 8

        def _zero(c, carry):
            for u in range(UNROLL):
                bins_v[pl.ds((c * UNROLL + u) * L, L)] = zeros16
            return carry

        lax.fori_loop(0, _NBIN_ROWS * K // L // UNROLL, _zero, 0)

        for j in range(rows_per_w):
            cp.wait()
            pltpu.sync_copy(
                rows_v,
                out_hbm.at[pl.ds((wid * rows_per_w + j) * CHUNK, CHUNK)])
            if j + 1 < rows_per_w:
                cp = pltpu.async_copy(table_hbm.at[idx_v.at[j + 1]], rows_v,
                                      sem)

        lane = lax.iota(jnp.int32, L)
        base16 = jnp.bitwise_and(lane, _NBIN_ROWS - 1) * K
        ones16 = jnp.ones((L,), jnp.float32)
        group = lax.shift_right_logical(lane, 1)
        masks = [group == q for q in range(L // _NBIN_ROWS)]
        for j in range(rows_per_w):
            for g in range(CHUNK // L):
                idx16 = idx_v[j, pl.ds(g * L, L)] + base16
                for m in masks:
                    plsc.addupdate_scatter(bins_v, [idx16], ones16, mask=m)

        def _fold(c, carry):
            for u in range(UNROLL):
                off = (c * UNROLL + u) * L
                bins_v[pl.ds(off, L)] = (bins_v[pl.ds(off, L)]
                                         + bins_v[pl.ds(K + off, L)])
            return carry

        lax.fori_loop(0, K // L // UNROLL, _fold, 0)
        pltpu.sync_copy(bins_v.at[pl.ds(0, K)], hist_hbm.at[pl.ds(wid * K, K)])

    return gather(emb_pad, idx2d)


def _perp_body(parts_ref, out_ref):
    btot = 8192.0
    hist = jnp.sum(parts_ref[...], axis=0, keepdims=True)
    probs = hist / btot
    out_ref[0] = jnp.exp(-jnp.sum(probs * jnp.log(probs + 1e-10)))


def _perplexity(hist_parts):
    return pl.pallas_call(
        _perp_body,
        out_specs=pl.BlockSpec(memory_space=pltpu.SMEM),
        out_shape=jax.ShapeDtypeStruct((1,), jnp.float32),
    )(hist_parts)


def kernel(z, embedding):
    B, D = z.shape
    K = embedding.shape[0]
    zz = jnp.sum(z * z, axis=1, keepdims=True)
    ee = jnp.sum(embedding * embedding, axis=1)
    indices, stats = _vq_main(z, embedding, zz, ee)
    emb_pad = jnp.pad(embedding, ((0, 0), (0, 128 - D)))
    z_q_pad, hist_parts = _sc_gather_hist(
        emb_pad, indices.reshape(B // 128, 128), B, K)
    perp = _perplexity(hist_parts.reshape(-1, K))
    return (z_q_pad[:, :D], stats[0], perp[0], indices, stats[1])
